# baseline (speedup 1.0000x reference)
import functools

import jax
import jax.numpy as jnp
from jax import lax
from jax.experimental import pallas as pl
from jax.experimental.pallas import tpu as pltpu
from jax.experimental.pallas import tpu_sc as plsc

D = 128
LANES = 16
NC, NS = 2, 16
NW = NC * NS
E_TOTAL = 320000
N_NODES = 10000
EPW = E_TOTAL // NW
CHUNK = 40
ROW = 2 * CHUNK
NCHUNK = EPW // CHUNK
GRP = 10
NGRP = NCHUNK // GRP
STAGE = 9984 // NS


def _build_kernel():
    mesh = plsc.VectorSubcoreMesh(core_axis_name="c", subcore_axis_name="s")

    @functools.partial(
        pl.kernel,
        mesh=mesh,
        out_type=jax.ShapeDtypeStruct((E_TOTAL, D), jnp.float32),
        scratch_types=[
            pltpu.VMEM((2 * GRP, ROW), jnp.int32),
            pltpu.VMEM((ROW, D), jnp.float32),
            pltpu.VMEM((ROW, D), jnp.float32),
            pltpu.VMEM((ROW, D), jnp.float32),
            pltpu.VMEM_SHARED((N_NODES, D), jnp.float32),
            pltpu.SemaphoreType.DMA,
            pltpu.SemaphoreType.DMA,
            pltpu.SemaphoreType.DMA,
            pltpu.SemaphoreType.DMA,
            pltpu.SemaphoreType.DMA,
            pltpu.SemaphoreType.DMA,
        ],
    )
    def had_kernel(h_hbm, idx_hbm, out_hbm,
                   gidx, gbuf0, gbuf1, gbuf2,
                   h_sp, gsem0, gsem1, gsem2, wsem0, wsem1, wsem2):
        wid = lax.axis_index("s") * NC + lax.axis_index("c")
        tid = lax.axis_index("s")
        gbuf = (gbuf0, gbuf1, gbuf2)
        gsem = (gsem0, gsem1, gsem2)
        wsem = (wsem0, wsem1, wsem2)

        pltpu.sync_copy(h_hbm.at[pl.ds(tid * STAGE, STAGE)],
                        h_sp.at[pl.ds(tid * STAGE, STAGE)])

        @pl.when(tid == 0)
        def _tail():
            pltpu.sync_copy(h_hbm.at[pl.ds(NS * STAGE, N_NODES - NS * STAGE)],
                            h_sp.at[pl.ds(NS * STAGE, N_NODES - NS * STAGE)])

        def load_group(g):
            pltpu.sync_copy(idx_hbm.at[wid, g],
                            gidx.at[pl.ds((g % 2) * GRP, GRP)])

        load_group(0)
        plsc.subcore_barrier()

        def table_for(s):
            return h_hbm if s == 2 else h_sp

        def fire_gather(c, s):
            pltpu.async_copy(table_for(s).at[gidx.at[c % (2 * GRP)]],
                             gbuf[s], gsem[s])

        def wait_gather(s):
            pltpu.make_async_copy(table_for(s).at[gidx.at[0]], gbuf[s],
                                  gsem[s]).wait()

        def multiply(s):
            def row_body(e, carry2):
                for d in range(D // LANES):
                    sl = pl.ds(d * LANES, LANES)
                    gbuf[s][e, sl] = gbuf[s][e, sl] * gbuf[s][e + CHUNK, sl]
                return carry2

            lax.fori_loop(0, CHUNK, row_body, 0, unroll=False)

        def fire_wb(c, s):
            off = wid * EPW + c * CHUNK
            pltpu.async_copy(gbuf[s].at[pl.ds(0, CHUNK)],
                             out_hbm.at[pl.ds(off, CHUNK)], wsem[s])

        def wait_wb(c, s):
            off = wid * EPW + c * CHUNK
            pltpu.make_async_copy(gbuf[s].at[pl.ds(0, CHUNK)],
                                  out_hbm.at[pl.ds(off, CHUNK)],
                                  wsem[s]).wait()

        def maybe_load_then_fire(c, s):
            @pl.when(c + 2 < NCHUNK)
            def _():
                @pl.when((c + 2) % GRP == 0)
                def _load():
                    load_group((c + 2) // GRP)

                fire_gather(c + 2, s)

        fire_gather(0, 0)
        fire_gather(1, 1)
        wait_gather(0)
        maybe_load_then_fire(0, 2)
        multiply(0)
        fire_wb(0, 0)
        wait_gather(1)
        wait_wb(0, 0)
        maybe_load_then_fire(1, 0)
        multiply(1)
        fire_wb(1, 1)

        def trio_body(i, carry):
            for b in range(3):
                c = 2 + i * 3 + b
                s = (2 + b) % 3
                wait_gather(s)
                wait_wb(c - 1, (1 + b) % 3)
                maybe_load_then_fire(c, (1 + b) % 3)
                multiply(s)
                fire_wb(c, s)
            return carry

        lax.fori_loop(0, (NCHUNK - 4) // 3, trio_body, 0, unroll=False)

        for c in (NCHUNK - 2, NCHUNK - 1):
            s = c % 3
            wait_gather(s)
            wait_wb(c - 1, (c - 1) % 3)
            multiply(s)
            fire_wb(c, s)

        wait_wb(NCHUNK - 1, (NCHUNK - 1) % 3)

    return had_kernel


_had_kernel = _build_kernel()


@jax.jit
def kernel(h, edge_label_index):
    ei = edge_label_index.astype(jnp.int32)
    src = ei[0].reshape(NW, NCHUNK, CHUNK)
    dst = ei[1].reshape(NW, NCHUNK, CHUNK)
    comb = jnp.concatenate([src, dst], axis=-1)
    comb = comb.reshape(NW, NGRP, GRP, ROW)
    return _had_kernel(h, comb)


# speedup vs baseline: 1.1429x; 1.1429x over previous
import functools

import jax
import jax.numpy as jnp
from jax import lax
from jax.experimental import pallas as pl
from jax.experimental.pallas import tpu as pltpu
from jax.experimental.pallas import tpu_sc as plsc

D = 128
LANES = 16
NC, NS = 2, 16
NW = NC * NS
E_TOTAL = 320000
N_NODES = 10000
EPW = E_TOTAL // NW
CHUNK = 40
ROW = 2 * CHUNK
NCHUNK = EPW // CHUNK
GRP = 10
NGRP = NCHUNK // GRP
STAGE = 9984 // NS


def _build_kernel():
    mesh = plsc.VectorSubcoreMesh(core_axis_name="c", subcore_axis_name="s")

    @functools.partial(
        pl.kernel,
        mesh=mesh,
        out_type=jax.ShapeDtypeStruct((E_TOTAL, D), jnp.float32),
        scratch_types=[
            pltpu.VMEM((2 * GRP, ROW), jnp.int32),
            pltpu.VMEM((ROW, D), jnp.float32),
            pltpu.VMEM((ROW, D), jnp.float32),
            pltpu.VMEM((ROW, D), jnp.float32),
            pltpu.VMEM((ROW, D), jnp.float32),
            pltpu.VMEM_SHARED((N_NODES, D), jnp.float32),
            pltpu.SemaphoreType.DMA,
            pltpu.SemaphoreType.DMA,
            pltpu.SemaphoreType.DMA,
            pltpu.SemaphoreType.DMA,
            pltpu.SemaphoreType.DMA,
            pltpu.SemaphoreType.DMA,
            pltpu.SemaphoreType.DMA,
            pltpu.SemaphoreType.DMA,
        ],
    )
    def had_kernel(h_hbm, idx_hbm, out_hbm,
                   gidx, gbuf0, gbuf1, gbuf2, gbuf3,
                   h_sp, gsem0, gsem1, gsem2, gsem3,
                   wsem0, wsem1, wsem2, wsem3):
        wid = lax.axis_index("s") * NC + lax.axis_index("c")
        tid = lax.axis_index("s")
        gbuf = (gbuf0, gbuf1, gbuf2, gbuf3)
        gsem = (gsem0, gsem1, gsem2, gsem3)
        wsem = (wsem0, wsem1, wsem2, wsem3)

        pltpu.sync_copy(h_hbm.at[pl.ds(tid * STAGE, STAGE)],
                        h_sp.at[pl.ds(tid * STAGE, STAGE)])

        @pl.when(tid == 0)
        def _tail():
            pltpu.sync_copy(h_hbm.at[pl.ds(NS * STAGE, N_NODES - NS * STAGE)],
                            h_sp.at[pl.ds(NS * STAGE, N_NODES - NS * STAGE)])

        def load_group(g):
            pltpu.sync_copy(idx_hbm.at[wid, g],
                            gidx.at[pl.ds((g % 2) * GRP, GRP)])

        load_group(0)
        plsc.subcore_barrier()

        def fire_gather(c, s):
            pltpu.async_copy(h_sp.at[gidx.at[c % (2 * GRP)]], gbuf[s],
                             gsem[s])

        def wait_gather(s):
            pltpu.make_async_copy(h_sp.at[gidx.at[0]], gbuf[s],
                                  gsem[s]).wait()

        def multiply(s):
            def row_body(e, carry2):
                for d in range(D // LANES):
                    sl = pl.ds(d * LANES, LANES)
                    gbuf[s][e, sl] = gbuf[s][e, sl] * gbuf[s][e + CHUNK, sl]
                return carry2

            lax.fori_loop(0, CHUNK, row_body, 0, unroll=False)

        def fire_wb(c, s):
            off = wid * EPW + c * CHUNK
            pltpu.async_copy(gbuf[s].at[pl.ds(0, CHUNK)],
                             out_hbm.at[pl.ds(off, CHUNK)], wsem[s])

        def wait_wb(c, s):
            off = wid * EPW + c * CHUNK
            pltpu.make_async_copy(gbuf[s].at[pl.ds(0, CHUNK)],
                                  out_hbm.at[pl.ds(off, CHUNK)],
                                  wsem[s]).wait()

        def maybe_load_then_fire(c, s):
            @pl.when(c + 2 < NCHUNK)
            def _():
                @pl.when((c + 2) % GRP == 0)
                def _load():
                    load_group((c + 2) // GRP)

                fire_gather(c + 2, s)

        fire_gather(0, 0)
        fire_gather(1, 1)
        wait_gather(0)
        maybe_load_then_fire(0, 2)
        multiply(0)
        fire_wb(0, 0)
        wait_gather(1)
        maybe_load_then_fire(1, 3)
        multiply(1)
        fire_wb(1, 1)

        def quad_body(i, carry):
            for b in range(4):
                c = 2 + i * 4 + b
                s = (2 + b) % 4
                wait_gather(s)
                wait_wb(c - 2, b)
                maybe_load_then_fire(c, b)
                multiply(s)
                fire_wb(c, s)
            return carry

        lax.fori_loop(0, (NCHUNK - 2) // 4, quad_body, 0, unroll=False)

        wait_wb(NCHUNK - 2, (NCHUNK - 2) % 4)
        wait_wb(NCHUNK - 1, (NCHUNK - 1) % 4)

    return had_kernel


_had_kernel = _build_kernel()


@jax.jit
def kernel(h, edge_label_index):
    ei = edge_label_index.astype(jnp.int32)
    src = ei[0].reshape(NW, NCHUNK, CHUNK)
    dst = ei[1].reshape(NW, NCHUNK, CHUNK)
    comb = jnp.concatenate([src, dst], axis=-1)
    comb = comb.reshape(NW, NGRP, GRP, ROW)
    return _had_kernel(h, comb)


# speedup vs baseline: 1.1532x; 1.0090x over previous
import functools

import jax
import jax.numpy as jnp
from jax import lax
from jax.experimental import pallas as pl
from jax.experimental.pallas import tpu as pltpu
from jax.experimental.pallas import tpu_sc as plsc

D = 128
LANES = 16
NC, NS = 2, 16
NW = NC * NS
E_TOTAL = 320000
N_NODES = 10000
EPW = E_TOTAL // NW
CHUNK = 40
ROW = 2 * CHUNK
NCHUNK = EPW // CHUNK
GRP = 10
NGRP = NCHUNK // GRP
STAGE = 9984 // NS


def _build_kernel():
    mesh = plsc.VectorSubcoreMesh(core_axis_name="c", subcore_axis_name="s")

    @functools.partial(
        pl.kernel,
        mesh=mesh,
        out_type=jax.ShapeDtypeStruct((E_TOTAL, D), jnp.float32),
        scratch_types=[
            pltpu.VMEM((2 * GRP, ROW), jnp.int32),
            pltpu.VMEM((ROW, D), jnp.float32),
            pltpu.VMEM((ROW, D), jnp.float32),
            pltpu.VMEM((ROW, D), jnp.float32),
            pltpu.VMEM((ROW, D), jnp.float32),
            pltpu.VMEM_SHARED((N_NODES, D), jnp.float32),
            pltpu.SemaphoreType.DMA,
            pltpu.SemaphoreType.DMA,
            pltpu.SemaphoreType.DMA,
            pltpu.SemaphoreType.DMA,
            pltpu.SemaphoreType.DMA,
            pltpu.SemaphoreType.DMA,
            pltpu.SemaphoreType.DMA,
            pltpu.SemaphoreType.DMA,
        ],
    )
    def had_kernel(h_hbm, idx_hbm, out_hbm,
                   gidx, gbuf0, gbuf1, gbuf2, gbuf3,
                   h_sp, gsem0, gsem1, gsem2, gsem3,
                   wsem0, wsem1, wsem2, wsem3):
        wid = lax.axis_index("s") * NC + lax.axis_index("c")
        tid = lax.axis_index("s")
        gbuf = (gbuf0, gbuf1, gbuf2, gbuf3)
        gsem = (gsem0, gsem1, gsem2, gsem3)
        wsem = (wsem0, wsem1, wsem2, wsem3)

        pltpu.sync_copy(h_hbm.at[pl.ds(tid * STAGE, STAGE)],
                        h_sp.at[pl.ds(tid * STAGE, STAGE)])

        @pl.when(tid == 0)
        def _tail():
            pltpu.sync_copy(h_hbm.at[pl.ds(NS * STAGE, N_NODES - NS * STAGE)],
                            h_sp.at[pl.ds(NS * STAGE, N_NODES - NS * STAGE)])

        def load_group(g):
            pltpu.sync_copy(idx_hbm.at[wid, g],
                            gidx.at[pl.ds((g % 2) * GRP, GRP)])

        load_group(0)
        plsc.subcore_barrier()

        def table_for(s):
            return h_hbm if s == 3 else h_sp

        def fire_gather(c, s):
            pltpu.async_copy(table_for(s).at[gidx.at[c % (2 * GRP)]],
                             gbuf[s], gsem[s])

        def wait_gather(s):
            pltpu.make_async_copy(table_for(s).at[gidx.at[0]], gbuf[s],
                                  gsem[s]).wait()

        def multiply(s):
            def row_body(e, carry2):
                for d in range(D // LANES):
                    sl = pl.ds(d * LANES, LANES)
                    gbuf[s][e, sl] = gbuf[s][e, sl] * gbuf[s][e + CHUNK, sl]
                return carry2

            lax.fori_loop(0, CHUNK, row_body, 0, unroll=False)

        def fire_wb(c, s):
            off = wid * EPW + c * CHUNK
            pltpu.async_copy(gbuf[s].at[pl.ds(0, CHUNK)],
                             out_hbm.at[pl.ds(off, CHUNK)], wsem[s])

        def wait_wb(c, s):
            off = wid * EPW + c * CHUNK
            pltpu.make_async_copy(gbuf[s].at[pl.ds(0, CHUNK)],
                                  out_hbm.at[pl.ds(off, CHUNK)],
                                  wsem[s]).wait()

        def maybe_load_then_fire(c, s):
            @pl.when(c + 2 < NCHUNK)
            def _():
                @pl.when((c + 2) % GRP == 0)
                def _load():
                    load_group((c + 2) // GRP)

                fire_gather(c + 2, s)

        fire_gather(0, 0)
        fire_gather(1, 1)
        wait_gather(0)
        maybe_load_then_fire(0, 2)
        multiply(0)
        fire_wb(0, 0)
        wait_gather(1)
        maybe_load_then_fire(1, 3)
        multiply(1)
        fire_wb(1, 1)

        def quad_body(i, carry):
            for b in range(4):
                c = 2 + i * 4 + b
                s = (2 + b) % 4
                wait_gather(s)
                wait_wb(c - 2, b)
                maybe_load_then_fire(c, b)
                multiply(s)
                fire_wb(c, s)
            return carry

        lax.fori_loop(0, (NCHUNK - 2) // 4, quad_body, 0, unroll=False)

        wait_wb(NCHUNK - 2, (NCHUNK - 2) % 4)
        wait_wb(NCHUNK - 1, (NCHUNK - 1) % 4)

    return had_kernel


_had_kernel = _build_kernel()


@jax.jit
def kernel(h, edge_label_index):
    ei = edge_label_index.astype(jnp.int32)
    src = ei[0].reshape(NW, NCHUNK, CHUNK)
    dst = ei[1].reshape(NW, NCHUNK, CHUNK)
    comb = jnp.concatenate([src, dst], axis=-1)
    comb = comb.reshape(NW, NGRP, GRP, ROW)
    return _had_kernel(h, comb)


# speedup vs baseline: 1.1553x; 1.0018x over previous
import functools

import jax
import jax.numpy as jnp
from jax import lax
from jax.experimental import pallas as pl
from jax.experimental.pallas import tpu as pltpu
from jax.experimental.pallas import tpu_sc as plsc

D = 128
LANES = 16
NC, NS = 2, 16
NW = NC * NS
E_TOTAL = 320000
N_NODES = 10000
EPW = E_TOTAL // NW
CHUNK = 40
ROW = 2 * CHUNK
NCHUNK = EPW // CHUNK
GRP = 10
NGRP = NCHUNK // GRP
STAGE = 9984 // NS


def _build_kernel():
    mesh = plsc.VectorSubcoreMesh(core_axis_name="c", subcore_axis_name="s")

    @functools.partial(
        pl.kernel,
        mesh=mesh,
        out_type=jax.ShapeDtypeStruct((E_TOTAL, D), jnp.float32),
        scratch_types=[
            pltpu.VMEM((2 * GRP, ROW), jnp.int32),
            pltpu.VMEM((ROW, D), jnp.float32),
            pltpu.VMEM((ROW, D), jnp.float32),
            pltpu.VMEM((ROW, D), jnp.float32),
            pltpu.VMEM((ROW, D), jnp.float32),
            pltpu.VMEM_SHARED((N_NODES, D), jnp.float32),
            pltpu.SemaphoreType.DMA,
            pltpu.SemaphoreType.DMA,
            pltpu.SemaphoreType.DMA,
            pltpu.SemaphoreType.DMA,
            pltpu.SemaphoreType.DMA,
            pltpu.SemaphoreType.DMA,
            pltpu.SemaphoreType.DMA,
            pltpu.SemaphoreType.DMA,
        ],
    )
    def had_kernel(h_hbm, idx_hbm, out_hbm,
                   gidx, gbuf0, gbuf1, gbuf2, gbuf3,
                   h_sp, gsem0, gsem1, gsem2, gsem3,
                   wsem0, wsem1, wsem2, wsem3):
        wid = lax.axis_index("s") * NC + lax.axis_index("c")
        tid = lax.axis_index("s")
        gbuf = (gbuf0, gbuf1, gbuf2, gbuf3)
        gsem = (gsem0, gsem1, gsem2, gsem3)
        wsem = (wsem0, wsem1, wsem2, wsem3)

        pltpu.sync_copy(h_hbm.at[pl.ds(tid * STAGE, STAGE)],
                        h_sp.at[pl.ds(tid * STAGE, STAGE)])

        @pl.when(tid == 0)
        def _tail():
            pltpu.sync_copy(h_hbm.at[pl.ds(NS * STAGE, N_NODES - NS * STAGE)],
                            h_sp.at[pl.ds(NS * STAGE, N_NODES - NS * STAGE)])

        def load_group(g):
            pltpu.sync_copy(idx_hbm.at[wid, g],
                            gidx.at[pl.ds((g % 2) * GRP, GRP)])

        load_group(0)
        plsc.subcore_barrier()

        def table_for(s):
            return h_hbm if s == 3 else h_sp

        def fire_gather(c, s):
            pltpu.async_copy(table_for(s).at[gidx.at[c % (2 * GRP)]],
                             gbuf[s], gsem[s])

        def wait_gather(s):
            pltpu.make_async_copy(table_for(s).at[gidx.at[0]], gbuf[s],
                                  gsem[s]).wait()

        def multiply(s):
            def row_body(e, carry2):
                for d in range(D // LANES):
                    sl = pl.ds(d * LANES, LANES)
                    gbuf[s][e, sl] = gbuf[s][e, sl] * gbuf[s][e + CHUNK, sl]
                return carry2

            lax.fori_loop(0, CHUNK, row_body, 0, unroll=2)

        def fire_wb(c, s):
            off = wid * EPW + c * CHUNK
            pltpu.async_copy(gbuf[s].at[pl.ds(0, CHUNK)],
                             out_hbm.at[pl.ds(off, CHUNK)], wsem[s])

        def wait_wb(c, s):
            off = wid * EPW + c * CHUNK
            pltpu.make_async_copy(gbuf[s].at[pl.ds(0, CHUNK)],
                                  out_hbm.at[pl.ds(off, CHUNK)],
                                  wsem[s]).wait()

        def maybe_load_then_fire(c, s):
            @pl.when(c + 2 < NCHUNK)
            def _():
                @pl.when((c + 2) % GRP == 0)
                def _load():
                    load_group((c + 2) // GRP)

                fire_gather(c + 2, s)

        fire_gather(0, 0)
        fire_gather(1, 1)
        wait_gather(0)
        maybe_load_then_fire(0, 2)
        multiply(0)
        fire_wb(0, 0)
        wait_gather(1)
        maybe_load_then_fire(1, 3)
        multiply(1)
        fire_wb(1, 1)

        def quad_body(i, carry):
            for b in range(4):
                c = 2 + i * 4 + b
                s = (2 + b) % 4
                wait_gather(s)
                wait_wb(c - 2, b)
                maybe_load_then_fire(c, b)
                multiply(s)
                fire_wb(c, s)
            return carry

        lax.fori_loop(0, (NCHUNK - 2) // 4, quad_body, 0, unroll=False)

        wait_wb(NCHUNK - 2, (NCHUNK - 2) % 4)
        wait_wb(NCHUNK - 1, (NCHUNK - 1) % 4)

    return had_kernel


_had_kernel = _build_kernel()


@jax.jit
def kernel(h, edge_label_index):
    ei = edge_label_index.astype(jnp.int32)
    src = ei[0].reshape(NW, NCHUNK, CHUNK)
    dst = ei[1].reshape(NW, NCHUNK, CHUNK)
    comb = jnp.concatenate([src, dst], axis=-1)
    comb = comb.reshape(NW, NGRP, GRP, ROW)
    return _had_kernel(h, comb)
